# baseline (speedup 1.0000x reference)
import functools

import jax
import jax.numpy as jnp
from jax import lax
from jax.experimental import pallas as pl
from jax.experimental.pallas import tpu as pltpu

K = 1024
D = 64
COMMITMENT_COST = 0.25
M_BLK = 2304


def _rownorm2(a):
    aa = a * a
    s8 = aa[:, 0:8]
    for k in range(1, 8):
        s8 = s8 + aa[:, 8 * k:8 * (k + 1)]
    s4 = s8[:, 0:4] + s8[:, 4:8]
    s2 = s4[:, 0:2] + s4[:, 2:4]
    return s2[:, 0:1] + s2[:, 1:2]


def _vq_block(x_ref, e_ref, zq_ref, idx_ref, loss_ref):
    x = x_ref[...]
    e = e_ref[...]
    x2 = _rownorm2(x)
    e2 = _rownorm2(e).reshape(1, K)
    xe = lax.dot_general(
        x, e, (((1,), (1,)), ((), ())),
        preferred_element_type=jnp.float32,
    )
    dist = jnp.sqrt(jnp.maximum(x2 + e2 - 2.0 * xe, 0.0))
    m = jnp.min(dist, axis=1, keepdims=True)
    cols = lax.broadcasted_iota(jnp.int32, dist.shape, 1)
    idx = jnp.min(jnp.where(dist == m, cols, K), axis=1)
    idx_ref[...] = idx[:, None]
    onehot = (cols == idx[:, None]).astype(jnp.float32)
    zq = lax.dot_general(
        onehot, e, (((1,), (0,)), ((), ())),
        preferred_element_type=jnp.float32,
    )
    zq_ref[...] = x + (zq - x)
    diff = zq - x
    part = (1.0 + COMMITMENT_COST) * jnp.sum(diff * diff)

    @pl.when(pl.program_id(0) == 0)
    def _():
        loss_ref[...] = jnp.zeros((1, 1), jnp.float32)

    loss_ref[...] += part.reshape(1, 1)


def kernel(z, embedding_weight):
    latents_shape = z.shape
    flat = z.reshape(-1, D)
    n = flat.shape[0]
    nb = n // M_BLK
    zq, idx, loss_parts = pl.pallas_call(
        _vq_block,
        grid=(nb,),
        in_specs=[
            pl.BlockSpec((M_BLK, D), lambda i: (i, 0)),
            pl.BlockSpec((K, D), lambda i: (0, 0)),
        ],
        out_specs=[
            pl.BlockSpec((M_BLK, D), lambda i: (i, 0)),
            pl.BlockSpec((M_BLK, 1), lambda i: (i, 0)),
            pl.BlockSpec((1, 1), lambda i: (0, 0)),
        ],
        out_shape=[
            jax.ShapeDtypeStruct((n, D), jnp.float32),
            jax.ShapeDtypeStruct((n, 1), jnp.int32),
            jax.ShapeDtypeStruct((1, 1), jnp.float32),
        ],
    )(flat, embedding_weight)
    return (
        zq.reshape(latents_shape),
        loss_parts[0, 0],
        idx.reshape(n),
    )
